# baseline (speedup 1.0000x reference)
import functools

import jax
import jax.numpy as jnp
from jax.experimental import pallas as pl
from jax.experimental.pallas import tpu as pltpu

_BR = 128
_BC = 32000


def _loss_kernel(t_ref, p_ref, o_ref, *, n_col_blocks, n_cols, block_cols):
    j = pl.program_id(1)
    x = p_ref[...]
    nls = jnp.log(1.0 + jnp.exp(-x))
    col = jax.lax.broadcasted_iota(jnp.int32, x.shape, 1) + j * block_cols
    nls = jnp.where(col == t_ref[...], 0.0, nls)
    part = jnp.sum(nls, axis=1, keepdims=True)

    @pl.when(j == 0)
    def _():
        o_ref[...] = part

    @pl.when(j > 0)
    def _():
        o_ref[...] = o_ref[...] + part

    @pl.when(j == n_col_blocks - 1)
    def _():
        o_ref[...] = o_ref[...] * (1.0 / n_cols)


def _build_call(b, c, interpret=False):
    grid = (b // _BR, c // _BC)
    return pl.pallas_call(
        functools.partial(
            _loss_kernel, n_col_blocks=grid[1], n_cols=c, block_cols=_BC
        ),
        out_shape=jax.ShapeDtypeStruct((b, 1), jnp.float32),
        grid=grid,
        in_specs=[
            pl.BlockSpec((_BR, 1), lambda i, j: (i, 0)),
            pl.BlockSpec((_BR, _BC), lambda i, j: (i, j)),
        ],
        out_specs=pl.BlockSpec((_BR, 1), lambda i, j: (i, 0)),
        compiler_params=pltpu.CompilerParams(
            dimension_semantics=("parallel", "arbitrary"),
            vmem_limit_bytes=58 * 1024 * 1024,
        ),
        name="adversarial_loss",
        interpret=interpret,
    )


def kernel(pred, target):
    b, c = pred.shape
    t = target.astype(jnp.int32).reshape(b, 1)
    out = _build_call(b, c)(t, pred)
    return out.reshape(b)


# speedup vs baseline: 1.0149x; 1.0149x over previous
import functools

import jax
import jax.numpy as jnp
from jax.experimental import pallas as pl
from jax.experimental.pallas import tpu as pltpu

_BR = 2048
_BC = 3200


def _loss_kernel(t_ref, p_ref, o_ref, *, n_col_blocks, n_cols, block_cols):
    j = pl.program_id(1)
    x = p_ref[...]
    nls = jnp.log(1.0 + jnp.exp(-x))
    col = jax.lax.broadcasted_iota(jnp.int32, x.shape, 1) + j * block_cols
    nls = jnp.where(col == t_ref[...], 0.0, nls)
    part = jnp.sum(nls, axis=1, keepdims=True)

    @pl.when(j == 0)
    def _():
        o_ref[...] = part

    @pl.when(j > 0)
    def _():
        o_ref[...] = o_ref[...] + part

    @pl.when(j == n_col_blocks - 1)
    def _():
        o_ref[...] = o_ref[...] * (1.0 / n_cols)


def _build_call(b, c, interpret=False):
    grid = (b // _BR, c // _BC)
    return pl.pallas_call(
        functools.partial(
            _loss_kernel, n_col_blocks=grid[1], n_cols=c, block_cols=_BC
        ),
        out_shape=jax.ShapeDtypeStruct((b, 1), jnp.float32),
        grid=grid,
        in_specs=[
            pl.BlockSpec((_BR, 1), lambda i, j: (i, 0)),
            pl.BlockSpec((_BR, _BC), lambda i, j: (i, j)),
        ],
        out_specs=pl.BlockSpec((_BR, 1), lambda i, j: (i, 0)),
        compiler_params=pltpu.CompilerParams(
            dimension_semantics=("parallel", "arbitrary"),
            vmem_limit_bytes=58 * 1024 * 1024,
        ),
        name="adversarial_loss",
        interpret=interpret,
    )


def kernel(pred, target):
    b, c = pred.shape
    t = target.astype(jnp.int32).reshape(b, 1)
    out = _build_call(b, c)(t, pred)
    return out.reshape(b)
